# baseline (speedup 1.0000x reference)
import jax
import jax.numpy as jnp
from jax.experimental import pallas as pl
from jax.experimental.pallas import tpu as pltpu

_COMBOS = (
    (0, 0.6, 1.0, 0, 0.0, 1.0, 5.0),
    (0, 0.5, 0.7, 2, 0.3, 0.7, 1.0),
    (0, 0.5, 0.7, 3, 0.2, 0.5, 1.0),
    (0, 0.5, 0.7, 5, 0.2, 0.4, 1.5),
    (0, 0.5, 0.7, 132, 0.2, 0.5, 1.0),
    (0, 0.5, 0.7, 36, 0.1, 0.3, 1.2),
)
_F = 42
_G = 5
_W = _F - _G + 1
_MIN_VALID = 3
_NEG = -1.0e30


def _sc_kernel(x_ref, out_j_ref, out_s_ref):
    combo_j = []
    for (ca, lo_a, hi_a, cb, lo_b, hi_b, _s) in _COMBOS:
        a = x_ref[:, ca:ca + 1]
        b = x_ref[:, cb:cb + 1]
        combo_j.append((a >= lo_a) & (a <= hi_a) & (b >= lo_b) & (b <= hi_b))

    frame_s = jnp.zeros((_F, 1), jnp.float32)
    for ((_ca, _la, _ha, _cb, _lb, _hb, s), cj) in reversed(
            list(zip(_COMBOS, combo_j))):
        frame_s = jnp.where(cj, jnp.float32(s), frame_s)
    frame_j = combo_j[0]
    for cj in combo_j[1:]:
        frame_j = frame_j | cj
    fj = frame_j.astype(jnp.float32)

    counts = fj[0:_W, :]
    sums = frame_s[0:_W, :]
    for k in range(1, _G):
        counts = counts + fj[k:k + _W, :]
        sums = sums + frame_s[k:k + _W, :]

    grp_j = counts >= jnp.float32(_MIN_VALID)
    masked = jnp.where(grp_j, sums, jnp.float32(_NEG))
    true_count = jnp.sum(grp_j.astype(jnp.float32))

    m1 = jnp.max(masked)
    iota = jax.lax.broadcasted_iota(jnp.int32, (_W, 1), 0)
    idx1 = jnp.min(jnp.where(masked == m1, iota, jnp.int32(_W)))
    m2 = jnp.max(jnp.where(iota == idx1, jnp.float32(_NEG), masked))

    final_j = true_count >= 2.0
    out_j_ref[0, 0] = final_j.astype(jnp.int32)
    out_s_ref[0, 0] = jnp.where(final_j, m1 + m2, 0.0).astype(jnp.float32)


@jax.jit
def kernel(speech_result):
    out_j, out_s = pl.pallas_call(
        _sc_kernel,
        out_shape=(
            jax.ShapeDtypeStruct((1, 1), jnp.int32),
            jax.ShapeDtypeStruct((1, 1), jnp.float32),
        ),
        in_specs=[pl.BlockSpec(memory_space=pltpu.VMEM)],
        out_specs=(
            pl.BlockSpec(memory_space=pltpu.SMEM),
            pl.BlockSpec(memory_space=pltpu.SMEM),
        ),
    )(speech_result)
    return out_j[0, 0] != 0, out_s[0, 0]


# speedup vs baseline: 1.0367x; 1.0367x over previous
import jax
import jax.numpy as jnp
from jax.experimental import pallas as pl
from jax.experimental.pallas import tpu as pltpu

_COMBOS = (
    (0, 0.6, 1.0, 0, 0.0, 1.0, 5.0),
    (0, 0.5, 0.7, 2, 0.3, 0.7, 1.0),
    (0, 0.5, 0.7, 3, 0.2, 0.5, 1.0),
    (0, 0.5, 0.7, 5, 0.2, 0.4, 1.5),
    (0, 0.5, 0.7, 132, 0.2, 0.5, 1.0),
    (0, 0.5, 0.7, 36, 0.1, 0.3, 1.2),
)
_F = 42
_G = 5
_W = _F - _G + 1
_MIN_VALID = 3
_NEG = -1.0e30

def _tc_kernel(x_ref, out_j_ref, out_s_ref):
    t_a = jnp.transpose(x_ref[:, 0:8])
    t_b = jnp.transpose(x_ref[:, 32:40])
    t_c = jnp.transpose(x_ref[:, 128:136])
    lanes = {
        0: t_a[0:1, :], 2: t_a[2:3, :], 3: t_a[3:4, :], 5: t_a[5:6, :],
        36: t_b[4:5, :], 132: t_c[4:5, :],
    }

    combo_j = []
    for (ca, lo_a, hi_a, cb, lo_b, hi_b, _s) in _COMBOS:
        a = lanes[ca]
        b = lanes[cb]
        combo_j.append((a >= lo_a) & (a <= hi_a) & (b >= lo_b) & (b <= hi_b))

    frame_s = jnp.zeros((1, _F), jnp.float32)
    for ((_ca, _la, _ha, _cb, _lb, _hb, s), cj) in reversed(
            list(zip(_COMBOS, combo_j))):
        frame_s = jnp.where(cj, jnp.float32(s), frame_s)
    frame_j = combo_j[0]
    for cj in combo_j[1:]:
        frame_j = frame_j | cj
    fj = frame_j.astype(jnp.float32)

    counts = fj[:, 0:_W]
    sums = frame_s[:, 0:_W]
    for k in range(1, _G):
        counts = counts + fj[:, k:k + _W]
        sums = sums + frame_s[:, k:k + _W]

    grp_j = counts >= jnp.float32(_MIN_VALID)
    masked = jnp.where(grp_j, sums, jnp.float32(_NEG))
    true_count = jnp.sum(grp_j.astype(jnp.float32), axis=1, keepdims=True)

    m1 = jnp.max(masked, axis=1, keepdims=True)
    eq = masked == jnp.broadcast_to(m1, (1, _W))
    n_at_max = jnp.sum(eq.astype(jnp.float32), axis=1, keepdims=True)
    m2_rest = jnp.max(jnp.where(eq, jnp.float32(_NEG), masked),
                      axis=1, keepdims=True)
    m2 = jnp.where(n_at_max >= 2.0, m1, m2_rest)

    tc_s = true_count[0, 0]
    top2_s = (m1 + m2)[0, 0]
    final_j = tc_s >= 2.0
    out_j_ref[...] = final_j
    out_s_ref[...] = jnp.where(final_j, top2_s, 0.0).astype(jnp.float32)


@jax.jit
def kernel(speech_result):
    out_j, out_s = pl.pallas_call(
        _tc_kernel,
        out_shape=(
            jax.ShapeDtypeStruct((), jnp.bool_),
            jax.ShapeDtypeStruct((), jnp.float32),
        ),
        in_specs=[pl.BlockSpec(memory_space=pltpu.VMEM)],
        out_specs=(
            pl.BlockSpec(memory_space=pltpu.SMEM),
            pl.BlockSpec(memory_space=pltpu.SMEM),
        ),
    )(speech_result)
    return out_j, out_s
